# baseline (speedup 1.0000x reference)
import functools

import jax
import jax.numpy as jnp
from jax import lax
from jax.experimental import pallas as pl
from jax.experimental.pallas import tpu as pltpu
from jax.experimental.pallas import tpu_sc as plsc

N = 10000
E = 160000
D_IN = 128
H1 = 64
H2 = 16

_E_PAD = 163840
_N_CHUNKS = _E_PAD // 128
_CPT = _N_CHUNKS // 32
_ACC_ROWS = N + 16
_RPT = 624
_TAIL = _ACC_ROWS - 16 * _RPT


_UROWS = 1
_UNITS = _CPT // _UROWS
_NBUF = 8
_GROUPS = _UNITS // _NBUF


def _make_propagate(width):
  mesh = plsc.VectorSubcoreMesh(core_axis_name="c", subcore_axis_name="s")

  @functools.partial(
      pl.kernel,
      mesh=mesh,
      out_type=jax.ShapeDtypeStruct((2, N, width), jnp.float32),
      compiler_params=pltpu.CompilerParams(use_tc_tiling_on_sc=False),
      scratch_types=[
          pltpu.VMEM((_CPT, 128), jnp.int32),
          pltpu.VMEM((_CPT, 128), jnp.int32),
      ] + [pltpu.VMEM((_UROWS * 128, width), jnp.float32)
           for _ in range(_NBUF)]
      + [pltpu.VMEM_SHARED((_ACC_ROWS, width), jnp.float32)]
      + [pltpu.SemaphoreType.DMA] * (2 * _NBUF),
  )
  def prop(table_hbm, edges_hbm, zeros_hbm, out_hbm,
           src_v, dst_v, *rest):
    bufs = rest[:_NBUF]
    acc_sh = rest[_NBUF]
    gsems = rest[_NBUF + 1:2 * _NBUF + 1]
    ssems = rest[2 * _NBUF + 1:]
    c = lax.axis_index("c")
    s = lax.axis_index("s")
    wid = s * 2 + c

    pltpu.sync_copy(edges_hbm.at[0, pl.ds(wid * _CPT, _CPT)], src_v)
    pltpu.sync_copy(edges_hbm.at[1, pl.ds(wid * _CPT, _CPT)], dst_v)

    pltpu.sync_copy(zeros_hbm.at[pl.ds(0, _RPT)],
                    acc_sh.at[pl.ds(s * _RPT, _RPT)])

    @pl.when(s == 15)
    def _():
      pltpu.sync_copy(zeros_hbm.at[pl.ds(0, _TAIL)],
                      acc_sh.at[pl.ds(16 * _RPT, _TAIL)])

    plsc.subcore_barrier()

    def fire_gather(u, b):
      pltpu.make_async_copy(
          table_hbm.at[src_v.at[u]], bufs[b], gsems[b]).start()

    def wait_gather(b):
      pltpu.make_async_copy(
          table_hbm.at[src_v.at[0]], bufs[b], gsems[b]).wait()

    def fire_scatter(u, b):
      pltpu.make_async_copy(
          bufs[b], acc_sh.at[dst_v.at[u]], ssems[b]).start(add=True)

    def wait_scatter(b):
      pltpu.make_async_copy(
          bufs[b], acc_sh.at[dst_v.at[0]], ssems[b]).wait()

    for b in range(_NBUF):
      fire_gather(b, b)

    def group(k, _):
      u0 = k * _NBUF
      for b in range(_NBUF):
        wait_gather(b)
        fire_scatter(u0 + b, b)
      for b in range(_NBUF):
        wait_scatter(b)

        @pl.when(k + 1 < _GROUPS)
        def _(b=b):
          fire_gather(u0 + _NBUF + b, b)
      return 0

    lax.fori_loop(0, _GROUPS, group, 0)
    plsc.subcore_barrier()

    pltpu.sync_copy(acc_sh.at[pl.ds(s * _RPT, _RPT)],
                    out_hbm.at[c, pl.ds(s * _RPT, _RPT)])

    @pl.when(s == 15)
    def _():
      pltpu.sync_copy(acc_sh.at[pl.ds(16 * _RPT, N - 16 * _RPT)],
                      out_hbm.at[c, pl.ds(16 * _RPT, N - 16 * _RPT)])

  return prop


_propagate_64 = _make_propagate(H1)
_propagate_32 = _make_propagate(2 * H2)


_BN = 1000


def _mm_kernel(x_ref, w_ref, o_ref):
  o_ref[...] = jnp.dot(x_ref[...], w_ref[...],
                       preferred_element_type=jnp.float32)


def _matmul_xw0(x, w0):
  return pl.pallas_call(
      _mm_kernel,
      grid=(N // _BN,),
      in_specs=[
          pl.BlockSpec((_BN, D_IN), lambda i: (i, 0)),
          pl.BlockSpec((D_IN, H1), lambda i: (0, 0)),
      ],
      out_specs=pl.BlockSpec((_BN, H1), lambda i: (i, 0)),
      out_shape=jax.ShapeDtypeStruct((N, H1), jnp.float32),
  )(x, w0)


def _relu_mm_kernel(p_ref, w1_ref, w2_ref, o_ref):
  h = jax.nn.relu(p_ref[0] + p_ref[1])
  w = jnp.concatenate([w1_ref[...], w2_ref[...]], axis=1)
  o_ref[...] = jnp.dot(h, w, preferred_element_type=jnp.float32)


def _relu_sum_matmul(p1, w1, w2):
  return pl.pallas_call(
      _relu_mm_kernel,
      grid=(N // _BN,),
      in_specs=[
          pl.BlockSpec((2, _BN, H1), lambda i: (0, i, 0)),
          pl.BlockSpec((H1, H2), lambda i: (0, 0)),
          pl.BlockSpec((H1, H2), lambda i: (0, 0)),
      ],
      out_specs=pl.BlockSpec((_BN, 2 * H2), lambda i: (i, 0)),
      out_shape=jax.ShapeDtypeStruct((N, 2 * H2), jnp.float32),
  )(p1, w1, w2)


def _reparam_kernel(p_ref, eps_ref, o_ref):
  s = p_ref[0] + p_ref[1]
  zm = s[:, :H2]
  zl = s[:, H2:]
  o_ref[...] = zm + eps_ref[...] * jnp.exp(zl)


def _reparam(p2, eps):
  return pl.pallas_call(
      _reparam_kernel,
      grid=(N // _BN,),
      in_specs=[
          pl.BlockSpec((2, _BN, 2 * H2), lambda i: (0, i, 0)),
          pl.BlockSpec((_BN, H2), lambda i: (i, 0)),
      ],
      out_specs=pl.BlockSpec((_BN, H2), lambda i: (i, 0)),
      out_shape=jax.ShapeDtypeStruct((N, H2), jnp.float32),
  )(p2, eps)


def _decode_kernel(zr_ref, zc_ref, o_ref):
  o_ref[...] = lax.dot_general(
      zr_ref[...], zc_ref[...], (((1,), (1,)), ((), ())),
      preferred_element_type=jnp.float32)


_BM_DEC = 400


def _decode(z):
  return pl.pallas_call(
      _decode_kernel,
      grid=(N // _BM_DEC,),
      in_specs=[
          pl.BlockSpec((_BM_DEC, H2), lambda i: (i, 0)),
          pl.BlockSpec((N, H2), lambda i: (0, 0)),
      ],
      out_specs=pl.BlockSpec((_BM_DEC, N), lambda i: (i, 0)),
      out_shape=jax.ShapeDtypeStruct((N, N), jnp.float32),
      compiler_params=pltpu.CompilerParams(
          dimension_semantics=("parallel",)),
  )(z, z)


def kernel(x, edge_index, eps, W0, W1, W2):
  ei = edge_index.astype(jnp.int32)
  pad = jnp.broadcast_to(
      jnp.array([[0], [N]], jnp.int32), (2, _E_PAD - E))
  edges = jnp.concatenate([ei, pad], axis=1).reshape(2, _N_CHUNKS, 128)
  zeros64 = jnp.zeros((_RPT, H1), jnp.float32)
  zeros32 = jnp.zeros((_RPT, 2 * H2), jnp.float32)

  xw0 = _matmul_xw0(x, W0)
  p1 = _propagate_64(xw0, edges, zeros64)
  hw = _relu_sum_matmul(p1, W1, W2)
  p2 = _propagate_32(hw, edges, zeros32)
  z = _reparam(p2, eps)
  return _decode(z).reshape(-1)


# speedup vs baseline: 1.1685x; 1.1685x over previous
import functools

import jax
import jax.numpy as jnp
from jax import lax
from jax.experimental import pallas as pl
from jax.experimental.pallas import tpu as pltpu
from jax.experimental.pallas import tpu_sc as plsc

N = 10000
E = 160000
D_IN = 128
H1 = 64
H2 = 16

_E_PAD = 163840
_N_CHUNKS = _E_PAD // 128
_CPT = _N_CHUNKS // 32
_ACC_ROWS = N + 16
_RPT = 624
_TAIL = _ACC_ROWS - 16 * _RPT


_UROWS = 1
_UNITS = _CPT // _UROWS
_NBUF = 4
_GROUPS = _UNITS // _NBUF


def _make_propagate(width):
  mesh = plsc.VectorSubcoreMesh(core_axis_name="c", subcore_axis_name="s")

  @functools.partial(
      pl.kernel,
      mesh=mesh,
      out_type=jax.ShapeDtypeStruct((2, N, width), jnp.float32),
      compiler_params=pltpu.CompilerParams(use_tc_tiling_on_sc=False),
      scratch_types=[
          pltpu.VMEM((_CPT, 128), jnp.int32),
          pltpu.VMEM((_CPT, 128), jnp.int32),
      ] + [pltpu.VMEM((_UROWS * 128, width), jnp.float32)
           for _ in range(_NBUF)]
      + [pltpu.VMEM_SHARED((_ACC_ROWS, width), jnp.float32)]
      + [pltpu.VMEM_SHARED((N, width), jnp.float32)]
      + [pltpu.SemaphoreType.DMA] * (2 * _NBUF),
  )
  def prop(table_hbm, edges_hbm, zeros_hbm, out_hbm,
           src_v, dst_v, *rest):
    bufs = rest[:_NBUF]
    acc_sh = rest[_NBUF]
    tab_sh = rest[_NBUF + 1]
    gsems = rest[_NBUF + 2:2 * _NBUF + 2]
    ssems = rest[2 * _NBUF + 2:]
    c = lax.axis_index("c")
    s = lax.axis_index("s")
    wid = s * 2 + c

    pltpu.sync_copy(edges_hbm.at[0, pl.ds(wid * _CPT, _CPT)], src_v)
    pltpu.sync_copy(edges_hbm.at[1, pl.ds(wid * _CPT, _CPT)], dst_v)

    pltpu.sync_copy(table_hbm.at[pl.ds(s * _RPT, _RPT)],
                    tab_sh.at[pl.ds(s * _RPT, _RPT)])

    @pl.when(s == 15)
    def _():
      pltpu.sync_copy(table_hbm.at[pl.ds(16 * _RPT, N - 16 * _RPT)],
                      tab_sh.at[pl.ds(16 * _RPT, N - 16 * _RPT)])

    pltpu.sync_copy(zeros_hbm.at[pl.ds(0, _RPT)],
                    acc_sh.at[pl.ds(s * _RPT, _RPT)])

    @pl.when(s == 15)
    def _():
      pltpu.sync_copy(zeros_hbm.at[pl.ds(0, _TAIL)],
                      acc_sh.at[pl.ds(16 * _RPT, _TAIL)])

    plsc.subcore_barrier()

    def fire_gather(u, b):
      pltpu.make_async_copy(
          tab_sh.at[src_v.at[u]], bufs[b], gsems[b]).start()

    def wait_gather(b):
      pltpu.make_async_copy(
          tab_sh.at[src_v.at[0]], bufs[b], gsems[b]).wait()

    def fire_scatter(u, b):
      pltpu.make_async_copy(
          bufs[b], acc_sh.at[dst_v.at[u]], ssems[b]).start(add=True)

    def wait_scatter(b):
      pltpu.make_async_copy(
          bufs[b], acc_sh.at[dst_v.at[0]], ssems[b]).wait()

    for b in range(_NBUF):
      fire_gather(b, b)

    def group(k, _):
      u0 = k * _NBUF
      for b in range(_NBUF):
        wait_gather(b)
        fire_scatter(u0 + b, b)
      for b in range(_NBUF):
        wait_scatter(b)

        @pl.when(k + 1 < _GROUPS)
        def _(b=b):
          fire_gather(u0 + _NBUF + b, b)
      return 0

    lax.fori_loop(0, _GROUPS, group, 0)
    plsc.subcore_barrier()

    pltpu.sync_copy(acc_sh.at[pl.ds(s * _RPT, _RPT)],
                    out_hbm.at[c, pl.ds(s * _RPT, _RPT)])

    @pl.when(s == 15)
    def _():
      pltpu.sync_copy(acc_sh.at[pl.ds(16 * _RPT, N - 16 * _RPT)],
                      out_hbm.at[c, pl.ds(16 * _RPT, N - 16 * _RPT)])

  return prop


_propagate_64 = _make_propagate(H1)
_propagate_32 = _make_propagate(2 * H2)


_BN = 1000


def _mm_kernel(x_ref, w_ref, o_ref):
  o_ref[...] = jnp.dot(x_ref[...], w_ref[...],
                       preferred_element_type=jnp.float32)


def _matmul_xw0(x, w0):
  return pl.pallas_call(
      _mm_kernel,
      grid=(N // _BN,),
      in_specs=[
          pl.BlockSpec((_BN, D_IN), lambda i: (i, 0)),
          pl.BlockSpec((D_IN, H1), lambda i: (0, 0)),
      ],
      out_specs=pl.BlockSpec((_BN, H1), lambda i: (i, 0)),
      out_shape=jax.ShapeDtypeStruct((N, H1), jnp.float32),
  )(x, w0)


def _relu_mm_kernel(p_ref, w1_ref, w2_ref, o_ref):
  h = jax.nn.relu(p_ref[0] + p_ref[1])
  w = jnp.concatenate([w1_ref[...], w2_ref[...]], axis=1)
  o_ref[...] = jnp.dot(h, w, preferred_element_type=jnp.float32)


def _relu_sum_matmul(p1, w1, w2):
  return pl.pallas_call(
      _relu_mm_kernel,
      grid=(N // _BN,),
      in_specs=[
          pl.BlockSpec((2, _BN, H1), lambda i: (0, i, 0)),
          pl.BlockSpec((H1, H2), lambda i: (0, 0)),
          pl.BlockSpec((H1, H2), lambda i: (0, 0)),
      ],
      out_specs=pl.BlockSpec((_BN, 2 * H2), lambda i: (i, 0)),
      out_shape=jax.ShapeDtypeStruct((N, 2 * H2), jnp.float32),
  )(p1, w1, w2)


def _reparam_kernel(p_ref, eps_ref, o_ref):
  s = p_ref[0] + p_ref[1]
  zm = s[:, :H2]
  zl = s[:, H2:]
  o_ref[...] = zm + eps_ref[...] * jnp.exp(zl)


def _reparam(p2, eps):
  return pl.pallas_call(
      _reparam_kernel,
      grid=(N // _BN,),
      in_specs=[
          pl.BlockSpec((2, _BN, 2 * H2), lambda i: (0, i, 0)),
          pl.BlockSpec((_BN, H2), lambda i: (i, 0)),
      ],
      out_specs=pl.BlockSpec((_BN, H2), lambda i: (i, 0)),
      out_shape=jax.ShapeDtypeStruct((N, H2), jnp.float32),
  )(p2, eps)


def _decode_kernel(zr_ref, zc_ref, o_ref):
  o_ref[...] = lax.dot_general(
      zr_ref[...], zc_ref[...], (((1,), (1,)), ((), ())),
      preferred_element_type=jnp.float32)


_BM_DEC = 400


def _decode(z):
  return pl.pallas_call(
      _decode_kernel,
      grid=(N // _BM_DEC,),
      in_specs=[
          pl.BlockSpec((_BM_DEC, H2), lambda i: (i, 0)),
          pl.BlockSpec((N, H2), lambda i: (0, 0)),
      ],
      out_specs=pl.BlockSpec((_BM_DEC, N), lambda i: (i, 0)),
      out_shape=jax.ShapeDtypeStruct((N, N), jnp.float32),
      compiler_params=pltpu.CompilerParams(
          dimension_semantics=("parallel",)),
  )(z, z)


def kernel(x, edge_index, eps, W0, W1, W2):
  ei = edge_index.astype(jnp.int32)
  pad = jnp.broadcast_to(
      jnp.array([[0], [N]], jnp.int32), (2, _E_PAD - E))
  edges = jnp.concatenate([ei, pad], axis=1).reshape(2, _N_CHUNKS, 128)
  zeros64 = jnp.zeros((_RPT, H1), jnp.float32)
  zeros32 = jnp.zeros((_RPT, 2 * H2), jnp.float32)

  xw0 = _matmul_xw0(x, W0)
  p1 = _propagate_64(xw0, edges, zeros64)
  hw = _relu_sum_matmul(p1, W1, W2)
  p2 = _propagate_32(hw, edges, zeros32)
  z = _reparam(p2, eps)
  return _decode(z).reshape(-1)
